# baseline (speedup 1.0000x reference)
import jax
import jax.numpy as jnp
from jax import lax
from jax.experimental import pallas as pl
from jax.experimental.pallas import tpu as pltpu
from jax.experimental.pallas import tpu_sc as plsc

_SEQ = 200
_LAG = 50
_D = 64
_B = 1024
_NC = 2
_NS = 16
_NW = _NC * _NS
_BPW = _B // _NW
_NV = (_SEQ + 15) // 16
_SEQPAD = _NV * 16
_CHUNKS = ((0, 112), (112, 88))


def _body(x_hbm, w0, w1, w2, w3, w4, out_hbm,
          xv, xidx, cidx, rows0, rows1, c345,
          sem_x0, sem_x1, sem_g, sem_w0, sem_w1):
  cid = lax.axis_index("c")
  sid = lax.axis_index("s")
  wid = sid * _NC + cid
  b0 = wid * _BPW

  for k in range(_NV):
    s = lax.iota(jnp.int32, 16) + (16 * k)
    pf = jnp.minimum(jnp.maximum(s - (_SEQ - _LAG - 1), 0), _LAG)
    isf = jnp.where(s >= (_SEQ - _LAG), 1, 0).astype(jnp.int32)
    cidx[0, pl.ds(16 * k, 16)] = pf
    cidx[1, pl.ds(16 * k, 16)] = isf

  pltpu.sync_copy(w2, c345.at[:, 0, :])
  for t, w in ((0, w3), (1, w4)):
    tmp = (rows0, rows1)[t].at[0]
    for off, sz in _CHUNKS:
      pltpu.async_copy(w.at[cidx.at[t, pl.ds(off, sz)]],
                       tmp.at[pl.ds(off, sz)], sem_g)
    pltpu.make_async_copy(w.at[cidx.at[t, pl.ds(0, _SEQ)]], tmp, sem_g).wait()

  def pack_row(s, carry):
    for t in range(2):
      tmp = (rows0, rows1)[t].at[0]
      for k in range(_D // 16):
        c345[s, t + 1, pl.ds(16 * k, 16)] = tmp[s, pl.ds(16 * k, 16)]
    return carry

  lax.fori_loop(0, _SEQ, pack_row, 0)

  sem_w = (sem_w0, sem_w1)
  zeros16 = jnp.zeros((16,), jnp.int32)
  last16 = jnp.full((16,), _SEQ - 1, jnp.int32)

  sem_x = (sem_x0, sem_x1)

  def fire_x(i, p):
    i = jnp.minimum(i, _BPW - 1)
    pltpu.async_copy(x_hbm.at[b0 + i], xv.at[p], sem_x[p])

  def wait_x(p):
    pltpu.make_async_copy(x_hbm.at[b0], xv.at[p], sem_x[p]).wait()

  def deinterleave(p):
    pvec = jnp.full((16,), p, jnp.int32)
    for k in range(_NV):
      s = jnp.minimum(lax.iota(jnp.int32, 16) + (16 * k), last16)
      v0 = plsc.load_gather(xv, [pvec, s, zeros16])
      v1 = plsc.load_gather(xv, [pvec, s, zeros16 + 1])
      xidx[p, 0, pl.ds(16 * k, 16)] = v0
      xidx[p, 1, pl.ds(16 * k, 16)] = v1

  def fire_g(p):
    for off, sz in _CHUNKS:
      pltpu.async_copy(w0.at[xidx.at[p, 0, pl.ds(off, sz)]],
                       rows0.at[p, pl.ds(off, sz)], sem_g)
      pltpu.async_copy(w1.at[xidx.at[p, 1, pl.ds(off, sz)]],
                       rows1.at[p, pl.ds(off, sz)], sem_g)

  def wait_g(p):
    pltpu.make_async_copy(w0.at[xidx.at[p, 0, pl.ds(0, _SEQ)]],
                          rows0.at[p], sem_g).wait()
    pltpu.make_async_copy(w1.at[xidx.at[p, 1, pl.ds(0, _SEQ)]],
                          rows1.at[p], sem_g).wait()

  def fire_w(i, p):
    b = b0 + i
    pltpu.async_copy(rows0.at[p], out_hbm.at[b, :, 0, :], sem_w[p])
    pltpu.async_copy(rows1.at[p], out_hbm.at[b, :, 1, :], sem_w[p])
    pltpu.async_copy(c345, out_hbm.at[b, :, pl.ds(2, 3), :], sem_w[p])

  def wait_w(i, p):
    b = b0 + i
    pltpu.make_async_copy(rows0.at[p], out_hbm.at[b, :, 0, :], sem_w[p]).wait()
    pltpu.make_async_copy(rows1.at[p], out_hbm.at[b, :, 1, :], sem_w[p]).wait()
    pltpu.make_async_copy(c345, out_hbm.at[b, :, pl.ds(2, 3), :],
                          sem_w[p]).wait()

  def pair(j, first, last_pair):
    a = 2 * j
    b = 2 * j + 1
    wait_g(0)
    fire_w(a, 0)
    wait_x(1)
    deinterleave(1)
    fire_x(b + 2, 1)
    if not first:
      wait_w(b - 2, 1)
    fire_g(1)
    wait_g(1)
    fire_w(b, 1)
    if not last_pair:
      wait_x(0)
      deinterleave(0)
      fire_x(a + 4, 0)
    wait_w(a, 0)
    if not last_pair:
      fire_g(0)

  fire_x(0, 0)
  fire_x(1, 1)
  wait_x(0)
  deinterleave(0)
  fire_x(2, 0)
  fire_g(0)
  pair(0, True, False)
  lax.fori_loop(1, _BPW // 2 - 1,
                lambda j, c: (pair(j, False, False), c)[1], 0)
  pair(_BPW // 2 - 1, False, True)
  wait_x(1)
  wait_x(0)
  wait_w(_BPW - 1, 1)


def kernel(x, W0, W1, W2, W3, W4):
  mesh = plsc.VectorSubcoreMesh(core_axis_name="c", subcore_axis_name="s")
  run = pl.kernel(
      _body,
      out_type=jax.ShapeDtypeStruct((_B, _SEQ, 5, _D), jnp.float32),
      mesh=mesh,
      scratch_types=[
          pltpu.VMEM((2, _SEQ, 2), jnp.int32),
          pltpu.VMEM((2, 2, _SEQPAD), jnp.int32),
          pltpu.VMEM((2, _SEQPAD), jnp.int32),
          pltpu.VMEM((2, _SEQ, _D), jnp.float32),
          pltpu.VMEM((2, _SEQ, _D), jnp.float32),
          pltpu.VMEM((_SEQ, 3, _D), jnp.float32),
          pltpu.SemaphoreType.DMA,
          pltpu.SemaphoreType.DMA,
          pltpu.SemaphoreType.DMA,
          pltpu.SemaphoreType.DMA,
          pltpu.SemaphoreType.DMA,
      ],
      compiler_params=pltpu.CompilerParams(use_tc_tiling_on_sc=False,
                                           needs_layout_passes=False),
  )
  return run(x, W0, W1, W2, W3, W4)


# speedup vs baseline: 1.8256x; 1.8256x over previous
import jax
import jax.numpy as jnp
from jax import lax
from jax.experimental import pallas as pl
from jax.experimental.pallas import tpu as pltpu
from jax.experimental.pallas import tpu_sc as plsc

_SEQ = 200
_LAG = 50
_D = 64
_B = 1024
_NC = 2
_NS = 16
_NW = _NC * _NS
_BPW = _B // _NW
_H = 100
_L = 128


def _body(xprep, cprep, w0, w1, w2, w3, w4, out_hbm,
          xpv, cpv, rows0, rows1, c2, c3, c4,
          sem_x0, sem_x1, sem_g0, sem_g1, sem_w0, sem_w1):
  cid = lax.axis_index("c")
  sid = lax.axis_index("s")
  wid = sid * _NC + cid
  b0 = wid * _BPW

  sem_x = (sem_x0, sem_x1)
  sem_g = (sem_g0, sem_g1)
  sem_w = (sem_w0, sem_w1)
  cbufs = (c2, c3, c4)

  pltpu.sync_copy(cprep, cpv)
  for t, w in enumerate((w2, w3, w4)):
    for h in range(2):
      pltpu.async_copy(w.at[cpv.at[2 * t + h]], cbufs[t].at[h], sem_g[h])
  for t, w in enumerate((w2, w3, w4)):
    for h in range(2):
      pltpu.make_async_copy(w.at[cpv.at[2 * t + h]], cbufs[t].at[h],
                            sem_g[h]).wait()

  def fire_x(i, p):
    i = jnp.minimum(i, _BPW - 1)
    pltpu.async_copy(xprep.at[b0 + i], xpv.at[p], sem_x[p])

  def wait_x(p):
    pltpu.make_async_copy(xprep.at[b0], xpv.at[p], sem_x[p]).wait()

  def fire_g(bi, h):
    pltpu.async_copy(w0.at[xpv.at[bi, 0 + h]], rows0.at[h], sem_g[h])
    pltpu.async_copy(w1.at[xpv.at[bi, 2 + h]], rows1.at[h], sem_g[h])

  def wait_g(bi, h):
    pltpu.make_async_copy(w0.at[xpv.at[bi, 0 + h]], rows0.at[h],
                          sem_g[h]).wait()
    pltpu.make_async_copy(w1.at[xpv.at[bi, 2 + h]], rows1.at[h],
                          sem_g[h]).wait()

  def fire_w(i, h):
    b = b0 + i
    for j, buf in enumerate((rows0, rows1, c2, c3, c4)):
      pltpu.async_copy(buf.at[h, pl.ds(0, _H), :],
                       out_hbm.at[b, pl.ds(_H * h, _H), j, pl.ds(0, _D)],
                       sem_w[h])

  def wait_w(i, h):
    b = b0 + i
    for j, buf in enumerate((rows0, rows1, c2, c3, c4)):
      pltpu.make_async_copy(buf.at[h, pl.ds(0, _H), :],
                            out_hbm.at[b, pl.ds(_H * h, _H), j, pl.ds(0, _D)],
                            sem_w[h]).wait()

  def unit(i, pi, h, last_b):
    wait_g(pi, h)
    fire_w(i, h)
    if h == 1:
      fire_x(i + 2, pi)
    wait_w(i, h)
    if not last_b:
      if h == 0:
        wait_x(1 - pi)
      fire_g(1 - pi, h)

  def batch(i, pi, last_b=False):
    unit(i, pi, 0, last_b)
    unit(i, pi, 1, last_b)

  def pair(j, last=False):
    batch(2 * j, 0)
    batch(2 * j + 1, 1, last_b=last)

  fire_x(0, 0)
  fire_x(1, 1)
  wait_x(0)
  fire_g(0, 0)
  fire_g(0, 1)
  pair(0)
  lax.fori_loop(1, _BPW // 2 - 1, lambda j, c: (pair(j), c)[1], 0)
  pair(_BPW // 2 - 1, last=True)
  wait_x(0)
  wait_x(1)


def kernel(x, W0, W1, W2, W3, W4):
  x = x.astype(jnp.int32)
  tok = jnp.minimum(jnp.arange(_L, dtype=jnp.int32)[None, :]
                    + jnp.array([0, _H], jnp.int32)[:, None], _SEQ - 1)
  xg = jnp.take(x, tok.reshape(-1), axis=1).reshape(_B, 2, _L, 2)
  xprep = xg.transpose(0, 3, 1, 2).reshape(_B, 4, _L)
  pf = jnp.minimum(jnp.maximum(tok - (_SEQ - _LAG - 1), 0), _LAG)
  isf = (tok >= (_SEQ - _LAG)).astype(jnp.int32)
  cprep = jnp.concatenate([tok, pf, isf], axis=0)

  mesh = plsc.VectorSubcoreMesh(core_axis_name="c", subcore_axis_name="s")
  run = pl.kernel(
      _body,
      out_type=jax.ShapeDtypeStruct((_B, _SEQ, 8, 128), jnp.float32),
      mesh=mesh,
      scratch_types=[
          pltpu.VMEM((2, 4, _L), jnp.int32),
          pltpu.VMEM((6, _L), jnp.int32),
          pltpu.VMEM((2, _L, _D), jnp.float32),
          pltpu.VMEM((2, _L, _D), jnp.float32),
          pltpu.VMEM((2, _L, _D), jnp.float32),
          pltpu.VMEM((2, _L, _D), jnp.float32),
          pltpu.VMEM((2, _L, _D), jnp.float32),
          pltpu.SemaphoreType.DMA,
          pltpu.SemaphoreType.DMA,
          pltpu.SemaphoreType.DMA,
          pltpu.SemaphoreType.DMA,
          pltpu.SemaphoreType.DMA,
          pltpu.SemaphoreType.DMA,
      ],
      compiler_params=pltpu.CompilerParams(use_tc_tiling_on_sc=False,
                                           needs_layout_passes=False),
  )
  outp = run(xprep, cprep, W0, W1, W2, W3, W4)
  return lax.slice(outp, (0, 0, 0, 0), (_B, _SEQ, 5, _D))


# speedup vs baseline: 1.8614x; 1.0196x over previous
import jax
import jax.numpy as jnp
from jax import lax
from jax.experimental import pallas as pl
from jax.experimental.pallas import tpu as pltpu
from jax.experimental.pallas import tpu_sc as plsc

_SEQ = 200
_LAG = 50
_D = 64
_B = 1024
_NC = 2
_NS = 16
_NW = _NC * _NS
_BPW = _B // _NW
_H = 100
_L = 128


def _body(xprep, cprep, w0, w1, w2, w3, w4, out_hbm,
          xpv, cpv, rows0, rows1, c2, c3, c4,
          sem_x0, sem_x1, sem_g0, sem_g1,
          sem_w00, sem_w01, sem_w10, sem_w11):
  cid = lax.axis_index("c")
  sid = lax.axis_index("s")
  wid = sid * _NC + cid
  b0 = wid * _BPW

  sem_x = (sem_x0, sem_x1)
  sem_g = (sem_g0, sem_g1)
  sem_w = ((sem_w00, sem_w01), (sem_w10, sem_w11))
  cbufs = (c2, c3, c4)

  pltpu.sync_copy(cprep, cpv)
  for t, w in enumerate((w2, w3, w4)):
    for h in range(2):
      pltpu.async_copy(w.at[cpv.at[2 * t + h]], cbufs[t].at[h], sem_g[h])
  for t, w in enumerate((w2, w3, w4)):
    for h in range(2):
      pltpu.make_async_copy(w.at[cpv.at[2 * t + h]], cbufs[t].at[h],
                            sem_g[h]).wait()

  def fire_x(i, p):
    i = jnp.minimum(i, _BPW - 1)
    pltpu.async_copy(xprep.at[b0 + i], xpv.at[p], sem_x[p])

  def wait_x(p):
    pltpu.make_async_copy(xprep.at[b0], xpv.at[p], sem_x[p]).wait()

  def fire_g(bi, h, d):
    pltpu.async_copy(w0.at[xpv.at[bi, 0 + h]], rows0.at[h, d], sem_g[h])
    pltpu.async_copy(w1.at[xpv.at[bi, 2 + h]], rows1.at[h, d], sem_g[h])

  def wait_g(h, d):
    pltpu.make_async_copy(w0.at[xpv.at[0, 0 + h]], rows0.at[h, d],
                          sem_g[h]).wait()
    pltpu.make_async_copy(w1.at[xpv.at[0, 2 + h]], rows1.at[h, d],
                          sem_g[h]).wait()

  def bufs_at(h, d):
    return (rows0.at[h, d], rows1.at[h, d], c2.at[h], c3.at[h], c4.at[h])

  def fire_w(i, h, d):
    b = b0 + i
    for j, buf in enumerate(bufs_at(h, d)):
      pltpu.async_copy(buf.at[pl.ds(0, _H), :],
                       out_hbm.at[b, pl.ds(_H * h, _H), j, pl.ds(0, _D)],
                       sem_w[h][d])

  def wait_w(i, h, d):
    b = b0 + i
    for j, buf in enumerate(bufs_at(h, d)):
      pltpu.make_async_copy(buf.at[pl.ds(0, _H), :],
                            out_hbm.at[b, pl.ds(_H * h, _H), j, pl.ds(0, _D)],
                            sem_w[h][d]).wait()

  def unit(i, pi, h, first, last_b):
    wait_g(h, pi)
    fire_w(i, h, pi)
    if h == 1:
      fire_x(i + 2, pi)
    if not last_b:
      if h == 0:
        wait_x(1 - pi)
      if not first:
        wait_w(i - 1, h, 1 - pi)
      fire_g(1 - pi, h, 1 - pi)

  def batch(i, pi, first=False, last_b=False):
    unit(i, pi, 0, first, last_b)
    unit(i, pi, 1, first, last_b)

  def pair(j, first=False, last=False):
    batch(2 * j, 0, first=first)
    batch(2 * j + 1, 1, last_b=last)

  fire_x(0, 0)
  fire_x(1, 1)
  wait_x(0)
  fire_g(0, 0, 0)
  fire_g(0, 1, 0)
  pair(0, first=True)
  lax.fori_loop(1, _BPW // 2 - 1, lambda j, c: (pair(j), c)[1], 0)
  pair(_BPW // 2 - 1, last=True)
  for h in range(2):
    wait_w(_BPW - 2, h, 0)
    wait_w(_BPW - 1, h, 1)
  wait_x(0)
  wait_x(1)


def kernel(x, W0, W1, W2, W3, W4):
  x = x.astype(jnp.int32)
  tok = jnp.minimum(jnp.arange(_L, dtype=jnp.int32)[None, :]
                    + jnp.array([0, _H], jnp.int32)[:, None], _SEQ - 1)
  xg = jnp.take(x, tok.reshape(-1), axis=1).reshape(_B, 2, _L, 2)
  xprep = xg.transpose(0, 3, 1, 2).reshape(_B, 4, _L)
  pf = jnp.minimum(jnp.maximum(tok - (_SEQ - _LAG - 1), 0), _LAG)
  isf = (tok >= (_SEQ - _LAG)).astype(jnp.int32)
  cprep = jnp.concatenate([tok, pf, isf], axis=0)

  mesh = plsc.VectorSubcoreMesh(core_axis_name="c", subcore_axis_name="s")
  run = pl.kernel(
      _body,
      out_type=jax.ShapeDtypeStruct((_B, _SEQ, 8, 128), jnp.float32),
      mesh=mesh,
      scratch_types=[
          pltpu.VMEM((2, 4, _L), jnp.int32),
          pltpu.VMEM((6, _L), jnp.int32),
          pltpu.VMEM((2, 2, _L, _D), jnp.float32),
          pltpu.VMEM((2, 2, _L, _D), jnp.float32),
          pltpu.VMEM((2, _L, _D), jnp.float32),
          pltpu.VMEM((2, _L, _D), jnp.float32),
          pltpu.VMEM((2, _L, _D), jnp.float32),
          pltpu.SemaphoreType.DMA,
          pltpu.SemaphoreType.DMA,
          pltpu.SemaphoreType.DMA,
          pltpu.SemaphoreType.DMA,
          pltpu.SemaphoreType.DMA,
          pltpu.SemaphoreType.DMA,
          pltpu.SemaphoreType.DMA,
          pltpu.SemaphoreType.DMA,
      ],
      compiler_params=pltpu.CompilerParams(use_tc_tiling_on_sc=False,
                                           needs_layout_passes=False),
  )
  outp = run(xprep, cprep, W0, W1, W2, W3, W4)
  return lax.slice(outp, (0, 0, 0, 0), (_B, _SEQ, 5, _D))


# speedup vs baseline: 1.8671x; 1.0031x over previous
import jax
import jax.numpy as jnp
from jax import lax
from jax.experimental import pallas as pl
from jax.experimental.pallas import tpu as pltpu
from jax.experimental.pallas import tpu_sc as plsc

_SEQ = 200
_LAG = 50
_D = 64
_B = 1024
_NC = 2
_NS = 16
_NW = _NC * _NS
_BPW = _B // _NW
_H = 100
_L = 128


def _body(xprep, cprep, w0, w1, w2, w3, w4, g01_hbm, cb_hbm,
          xpv, cpv, rows0, rows1,
          sem_x0, sem_x1, sem_g0, sem_g1,
          sem_w00, sem_w01, sem_w10, sem_w11):
  cid = lax.axis_index("c")
  sid = lax.axis_index("s")
  wid = sid * _NC + cid
  b0 = wid * _BPW

  sem_x = (sem_x0, sem_x1)
  sem_g = (sem_g0, sem_g1)
  sem_w = ((sem_w00, sem_w01), (sem_w10, sem_w11))

  @pl.when(wid == 0)
  def _():
    pltpu.sync_copy(cprep, cpv)
    for t, w in enumerate((w2, w3, w4)):
      for h in range(2):
        pltpu.async_copy(w.at[cpv.at[2 * t + h]], rows0.at[h, 0], sem_g[h])
        pltpu.make_async_copy(w.at[cpv.at[2 * t + h]], rows0.at[h, 0],
                              sem_g[h]).wait()
        pltpu.sync_copy(rows0.at[h, 0, pl.ds(0, _H), :],
                        cb_hbm.at[pl.ds(_H * h, _H), pl.ds(128 * t, _D)])

  def fire_x(i, p):
    i = jnp.minimum(i, _BPW - 1)
    pltpu.async_copy(xprep.at[b0 + i], xpv.at[p], sem_x[p])

  def wait_x(p):
    pltpu.make_async_copy(xprep.at[b0], xpv.at[p], sem_x[p]).wait()

  def fire_g(bi, h, d):
    pltpu.async_copy(w0.at[xpv.at[bi, 0 + h]], rows0.at[h, d], sem_g[h])
    pltpu.async_copy(w1.at[xpv.at[bi, 2 + h]], rows1.at[h, d], sem_g[h])

  def wait_g(h, d):
    pltpu.make_async_copy(w0.at[xpv.at[0, 0 + h]], rows0.at[h, d],
                          sem_g[h]).wait()
    pltpu.make_async_copy(w1.at[xpv.at[0, 2 + h]], rows1.at[h, d],
                          sem_g[h]).wait()

  def fire_w(i, h, d):
    b = b0 + i
    for j, buf in enumerate((rows0, rows1)):
      pltpu.async_copy(buf.at[h, d, pl.ds(0, _H), :],
                       g01_hbm.at[b, pl.ds(_H * h, _H), pl.ds(_D * j, _D)],
                       sem_w[h][d])

  def wait_w(i, h, d):
    b = b0 + i
    for j, buf in enumerate((rows0, rows1)):
      pltpu.make_async_copy(buf.at[h, d, pl.ds(0, _H), :],
                            g01_hbm.at[b, pl.ds(_H * h, _H), pl.ds(_D * j, _D)],
                            sem_w[h][d]).wait()

  def unit(i, pi, h, first, last_b):
    wait_g(h, pi)
    fire_w(i, h, pi)
    if h == 1:
      fire_x(i + 2, pi)
    if not last_b:
      if h == 0:
        wait_x(1 - pi)
      if not first:
        wait_w(i - 1, h, 1 - pi)
      fire_g(1 - pi, h, 1 - pi)

  def batch(i, pi, first=False, last_b=False):
    unit(i, pi, 0, first, last_b)
    unit(i, pi, 1, first, last_b)

  def pair(j, first=False, last=False):
    batch(2 * j, 0, first=first)
    batch(2 * j + 1, 1, last_b=last)

  fire_x(0, 0)
  fire_x(1, 1)
  wait_x(0)
  fire_g(0, 0, 0)
  fire_g(0, 1, 0)
  pair(0, first=True)
  lax.fori_loop(1, _BPW // 2 - 1, lambda j, c: (pair(j), c)[1], 0)
  pair(_BPW // 2 - 1, last=True)
  for h in range(2):
    wait_w(_BPW - 2, h, 0)
    wait_w(_BPW - 1, h, 1)
  wait_x(0)
  wait_x(1)


def kernel(x, W0, W1, W2, W3, W4):
  x = x.astype(jnp.int32)
  tok = jnp.minimum(jnp.arange(_L, dtype=jnp.int32)[None, :]
                    + jnp.array([0, _H], jnp.int32)[:, None], _SEQ - 1)
  xg = jnp.take(x, tok.reshape(-1), axis=1).reshape(_B, 2, _L, 2)
  xprep = xg.transpose(0, 3, 1, 2).reshape(_B, 4, _L)
  pf = jnp.minimum(jnp.maximum(tok - (_SEQ - _LAG - 1), 0), _LAG)
  isf = (tok >= (_SEQ - _LAG)).astype(jnp.int32)
  cprep = jnp.concatenate([tok, pf, isf], axis=0)

  mesh = plsc.VectorSubcoreMesh(core_axis_name="c", subcore_axis_name="s")
  run = pl.kernel(
      _body,
      out_type=(jax.ShapeDtypeStruct((_B, _SEQ, 2 * _D), jnp.float32),
                jax.ShapeDtypeStruct((_SEQ, 3 * 128), jnp.float32)),
      mesh=mesh,
      scratch_types=[
          pltpu.VMEM((2, 4, _L), jnp.int32),
          pltpu.VMEM((6, _L), jnp.int32),
          pltpu.VMEM((2, 2, _L, _D), jnp.float32),
          pltpu.VMEM((2, 2, _L, _D), jnp.float32),
          pltpu.SemaphoreType.DMA,
          pltpu.SemaphoreType.DMA,
          pltpu.SemaphoreType.DMA,
          pltpu.SemaphoreType.DMA,
          pltpu.SemaphoreType.DMA,
          pltpu.SemaphoreType.DMA,
          pltpu.SemaphoreType.DMA,
          pltpu.SemaphoreType.DMA,
      ],
      compiler_params=pltpu.CompilerParams(use_tc_tiling_on_sc=False,
                                           needs_layout_passes=False),
  )
  g01, cblock = run(xprep, cprep, W0, W1, W2, W3, W4)
  cb = cblock.reshape(_SEQ, 3, 128)[:, :, :_D]
  return jnp.concatenate([
      g01[:, :, :_D][:, :, None, :],
      g01[:, :, _D:][:, :, None, :],
      jnp.broadcast_to(cb[None], (_B, _SEQ, 3, _D)),
  ], axis=2)
